# baseline (speedup 1.0000x reference)
import functools

import jax
import jax.numpy as jnp
import numpy as np
from jax import lax
from jax.experimental import pallas as pl
from jax.experimental.pallas import tpu as pltpu
from jax.experimental.pallas import tpu_sc as plsc

FIELD_DIMS = [100000] * 26
EMBED_DIM = 16
BATCH = 4096
NUM_FIELDS = len(FIELD_DIMS)

NC, NS, L = 2, 16, 16
NW = NC * NS
B_PER_W = BATCH // NW
ROWS_PER_W = B_PER_W * NUM_FIELDS
ELEMS_PER_W = ROWS_PER_W * EMBED_DIM
CHUNK = 128
NCHUNK_E = ELEMS_PER_W // CHUNK
NCHUNK_L = ROWS_PER_W // CHUNK
GROUPS = B_PER_W // L


def _fm_body(xie_hbm, xil_hbm, bias_hbm, emb_hbm, lin_hbm, out_hbm,
             idx_e, idx_l, rows_v, lin_v, out_v, bias_v, sem_e, sem_l):
  wid = lax.axis_index("s") * NC + lax.axis_index("c")

  pltpu.sync_copy(xie_hbm.at[wid], idx_e)
  pltpu.sync_copy(xil_hbm.at[wid], idx_l)
  pltpu.sync_copy(bias_hbm, bias_v)

  def fire_e(j, _):
    pltpu.make_async_copy(
        emb_hbm.at[idx_e.at[j]], rows_v.at[pl.ds(j * CHUNK, CHUNK)], sem_e
    ).start()
    return _

  lax.fori_loop(0, NCHUNK_E, fire_e, None)

  def fire_l(j, _):
    pltpu.make_async_copy(
        lin_hbm.at[idx_l.at[j]], lin_v.at[pl.ds(j * CHUNK, CHUNK)], sem_l
    ).start()
    return _

  lax.fori_loop(0, NCHUNK_L, fire_l, None)

  pltpu.make_async_copy(lin_hbm.at[pl.ds(0, ROWS_PER_W)], lin_v, sem_l).wait()
  pltpu.make_async_copy(emb_hbm.at[pl.ds(0, ELEMS_PER_W)], rows_v, sem_e).wait()

  lane = lax.iota(jnp.int32, L)
  bias_vec = bias_v[:]

  def group(g, _):
    acc = bias_vec
    for f in range(NUM_FIELDS):
      acc = acc + lin_v[pl.ds(f * B_PER_W + g * L, L)]

    gbase = g * (L * NUM_FIELDS)
    fmv = jnp.zeros((L,), jnp.float32)
    for j in range(L):
      base = (gbase + j * NUM_FIELDS) * EMBED_DIM
      r = rows_v[pl.ds(base, EMBED_DIM)]
      s = r
      sq = r * r
      for f in range(1, NUM_FIELDS):
        r = rows_v[pl.ds(base + f * EMBED_DIM, EMBED_DIM)]
        s = s + r
        sq = sq + r * r
      fm = 0.5 * jnp.sum(s * s - sq)
      fmv = jnp.where(lane == j, fm, fmv)

    out_v[pl.ds(g * L, L)] = acc + fmv
    return _

  lax.fori_loop(0, GROUPS, group, None)

  pltpu.sync_copy(out_v, out_hbm.at[pl.ds(wid * B_PER_W, B_PER_W)])


_fm_call = functools.partial(
    pl.kernel,
    out_type=jax.ShapeDtypeStruct((BATCH,), jnp.float32),
    mesh=plsc.VectorSubcoreMesh(core_axis_name="c", subcore_axis_name="s"),
    compiler_params=pltpu.CompilerParams(needs_layout_passes=False),
    scratch_types=[
        pltpu.VMEM((NCHUNK_E, CHUNK), jnp.int32),
        pltpu.VMEM((NCHUNK_L, CHUNK), jnp.int32),
        pltpu.VMEM((ELEMS_PER_W,), jnp.float32),
        pltpu.VMEM((ROWS_PER_W,), jnp.float32),
        pltpu.VMEM((B_PER_W,), jnp.float32),
        pltpu.VMEM((L,), jnp.float32),
        pltpu.SemaphoreType.DMA,
        pltpu.SemaphoreType.DMA,
    ],
)(_fm_body)

_OFFSETS = np.concatenate([[0], np.cumsum(FIELD_DIMS)[:-1]]).astype(np.int32)


def kernel(x, W_emb, W_lin, bias):
  xi = (x - 1) + jnp.asarray(_OFFSETS)[None, :]
  xie = xi.reshape(NW, ROWS_PER_W)[:, :, None] * EMBED_DIM + jnp.arange(
      EMBED_DIM, dtype=jnp.int32
  )
  xie = xie.reshape(NW, NCHUNK_E, CHUNK)
  xil = xi.reshape(NW, B_PER_W, NUM_FIELDS).transpose(0, 2, 1)
  xil = xil.reshape(NW, NCHUNK_L, CHUNK)
  bias16 = jnp.broadcast_to(bias, (L,)).astype(jnp.float32)
  return _fm_call(xie, xil, bias16, W_emb.reshape(-1), W_lin.reshape(-1))


# speedup vs baseline: 1.0818x; 1.0818x over previous
import functools

import jax
import jax.numpy as jnp
import numpy as np
from jax import lax
from jax.experimental import pallas as pl
from jax.experimental.pallas import tpu as pltpu
from jax.experimental.pallas import tpu_sc as plsc

FIELD_DIMS = [100000] * 26
EMBED_DIM = 16
BATCH = 4096
NUM_FIELDS = len(FIELD_DIMS)

NC, NS, L = 2, 16, 16
NW = NC * NS
B_PER_W = BATCH // NW
ROWS_PER_W = B_PER_W * NUM_FIELDS
ELEMS_PER_W = ROWS_PER_W * EMBED_DIM
CHUNK = 128
NCHUNK_E = ELEMS_PER_W // CHUNK
NCHUNK_L = ROWS_PER_W // CHUNK
GROUPS = B_PER_W // L
TOTAL_ROWS = sum(FIELD_DIMS)


def _fm_body(xie_hbm, xil_hbm, bias_hbm, emb_hbm, lin_hbm, out_hbm,
             idx_e, idx_l, rows_v, lin_v, out_v, bias_v, sem_e, sem_l):
  wid = lax.axis_index("s") * NC + lax.axis_index("c")

  pltpu.sync_copy(xie_hbm.at[wid], idx_e)
  pltpu.sync_copy(xil_hbm.at[wid], idx_l)
  pltpu.sync_copy(bias_hbm, bias_v)

  def fire_e(j, _):
    pltpu.make_async_copy(
        emb_hbm.at[idx_e.at[j]], rows_v.at[pl.ds(j * CHUNK, CHUNK)], sem_e
    ).start()
    return _

  lax.fori_loop(0, NCHUNK_L, fire_e, None)

  def fire_l(j, _):
    pltpu.make_async_copy(
        lin_hbm.at[idx_l.at[j]], lin_v.at[pl.ds(j * CHUNK, CHUNK)], sem_l
    ).start()
    return _

  lax.fori_loop(0, NCHUNK_L, fire_l, None)

  pltpu.make_async_copy(lin_hbm.at[pl.ds(0, ROWS_PER_W)], lin_v, sem_l).wait()
  pltpu.make_async_copy(emb_hbm.at[pl.ds(0, ROWS_PER_W)], rows_v, sem_e).wait()

  lane = lax.iota(jnp.int32, L)
  bias_vec = bias_v[:]

  def group(g, _):
    acc = bias_vec
    for f in range(NUM_FIELDS):
      acc = acc + lin_v[pl.ds(f * B_PER_W + g * L, L)]

    gbase = g * (L * NUM_FIELDS)
    fmv = jnp.zeros((L,), jnp.float32)
    for j in range(L):
      base = gbase + j * NUM_FIELDS
      r = rows_v[base, :]
      s = r
      sq = r * r
      for f in range(1, NUM_FIELDS):
        r = rows_v[base + f, :]
        s = s + r
        sq = sq + r * r
      fm = 0.5 * jnp.sum(s * s - sq)
      fmv = jnp.where(lane == j, fm, fmv)

    out_v[pl.ds(g * L, L)] = acc + fmv
    return _

  lax.fori_loop(0, GROUPS, group, None)

  pltpu.sync_copy(out_v, out_hbm.at[pl.ds(wid * B_PER_W, B_PER_W)])


_fm_call = functools.partial(
    pl.kernel,
    out_type=jax.ShapeDtypeStruct((BATCH,), jnp.float32),
    mesh=plsc.VectorSubcoreMesh(core_axis_name="c", subcore_axis_name="s"),
    compiler_params=pltpu.CompilerParams(
        needs_layout_passes=False, use_tc_tiling_on_sc=False
    ),
    scratch_types=[
        pltpu.VMEM((NCHUNK_L, CHUNK), jnp.int32),
        pltpu.VMEM((NCHUNK_L, CHUNK), jnp.int32),
        pltpu.VMEM((ROWS_PER_W, EMBED_DIM), jnp.float32),
        pltpu.VMEM((ROWS_PER_W,), jnp.float32),
        pltpu.VMEM((B_PER_W,), jnp.float32),
        pltpu.VMEM((L,), jnp.float32),
        pltpu.SemaphoreType.DMA,
        pltpu.SemaphoreType.DMA,
    ],
)(_fm_body)

_OFFSETS = np.concatenate([[0], np.cumsum(FIELD_DIMS)[:-1]]).astype(np.int32)


def kernel(x, W_emb, W_lin, bias):
  xi = (x - 1) + jnp.asarray(_OFFSETS)[None, :]
  xie = xi.reshape(NW, NCHUNK_L, CHUNK)
  xil = xi.reshape(NW, B_PER_W, NUM_FIELDS).transpose(0, 2, 1)
  xil = xil.reshape(NW, NCHUNK_L, CHUNK)
  bias16 = jnp.broadcast_to(bias, (L,)).astype(jnp.float32)
  return _fm_call(xie, xil, bias16, W_emb, W_lin.reshape(-1))
